# baseline (speedup 1.0000x reference)
import functools

import jax
import jax.numpy as jnp
from jax import lax
from jax.experimental import pallas as pl
from jax.experimental.pallas import tpu as pltpu
from jax.experimental.pallas import tpu_sc as plsc

F32 = jnp.float32

V = 10000
E = 160000
D = 16
NC = 2
NS = 16
NW = NC * NS
NH = 2
EH = E // NH
EW = EH // NW
B = 125
K = EW // B
VS = V // NS


def _proj_body(nf_ref, wp_ref, bp_ref, out_ref):
    for k in range(8):
        out_ref[:, D * k:D * (k + 1)] = jnp.maximum(
            jnp.dot(nf_ref[:, k, :], wp_ref[...], preferred_element_type=F32)
            + bp_ref[...], 0.0)


def _msg_body(ea_ref, xs_ref, we1_ref, be1_ref, we2_ref, be2_ref,
              r_ref, s_ref, out_ref, wout_ref):
    bf = jnp.bfloat16

    def mm(a, m_ref):
        return jnp.dot(a.astype(bf), m_ref[...], preferred_element_type=F32)

    zero = jnp.asarray(0.0, bf)
    e1 = jnp.maximum(mm(ea_ref[...], we1_ref).astype(bf) + be1_ref[...], zero)
    w = jnp.maximum(mm(e1, we2_ref).astype(bf) + be2_ref[...], zero)
    wout_ref[...] = w
    xs_rep = mm(xs_ref[...], r_ref).astype(bf)
    out_ref[...] = mm(w * xs_rep, s_ref)


def _msg_cached_body(xs_ref, w_ref, r_ref, s_ref, out_ref):
    bf = jnp.bfloat16

    def mm(a, m_ref):
        return jnp.dot(a.astype(bf), m_ref[...], preferred_element_type=F32)

    xs_rep = mm(xs_ref[...], r_ref).astype(bf)
    out_ref[...] = mm(w_ref[...] * xs_rep, s_ref)


def _update_body(pa_ref, pb_ref, x_ref, wr_ref, bc_ref,
                 wir_ref, wiz_ref, win_ref, bi_ref,
                 whr_ref, whz_ref, whn_ref, bh_ref, out_ref):

    def mm(a, m_ref):
        return jnp.dot(a, m_ref[...], preferred_element_type=F32)

    x = x_ref[...]
    agg = (pa_ref[0] + pa_ref[1]) + (pb_ref[0] + pb_ref[1])
    c = jnp.maximum(agg + mm(x, wr_ref) + bc_ref[...], 0.0)
    r = jax.nn.sigmoid(mm(c, wir_ref) + bi_ref[0:1, :]
                       + mm(x, whr_ref) + bh_ref[0:1, :])
    z = jax.nn.sigmoid(mm(c, wiz_ref) + bi_ref[1:2, :]
                       + mm(x, whz_ref) + bh_ref[1:2, :])
    n = jnp.tanh(mm(c, win_ref) + bi_ref[2:3, :]
                 + r * (mm(x, whn_ref) + bh_ref[2:3, :]))
    out_ref[...] = (1.0 - z) * n + z * x


def _gather_body(x_hbm, src_hbm, out_hbm, idx_v, rows_v, sem):
    c = lax.axis_index("c")
    s = lax.axis_index("s")
    wid = s * NC + c
    pltpu.sync_copy(src_hbm.at[wid], idx_v)

    def fire(j, carry):
        pltpu.async_copy(x_hbm.at[idx_v.at[j]], rows_v.at[j], sem)
        return carry

    lax.fori_loop(0, K, fire, 0)
    pltpu.make_async_copy(out_hbm.at[wid], rows_v, sem).wait()
    pltpu.sync_copy(rows_v, out_hbm.at[wid])


def _scatter_body(msg_hbm, dst_hbm, zeros_hbm, out_hbm,
                  msg_v, dst_v, buf_v, agg_sp, sem):
    c = lax.axis_index("c")
    s = lax.axis_index("s")
    wid = s * NC + c
    pltpu.sync_copy(zeros_hbm, buf_v)
    pltpu.sync_copy(buf_v, agg_sp.at[pl.ds(s * VS, VS)])
    pltpu.sync_copy(msg_hbm.at[wid], msg_v)
    pltpu.sync_copy(dst_hbm.at[wid], dst_v)
    plsc.subcore_barrier()

    def fire(j, carry):
        pltpu.async_copy(msg_v.at[j], agg_sp.at[dst_v.at[j]], sem, add=True)
        return carry

    lax.fori_loop(0, K, fire, 0)
    pltpu.make_async_copy(msg_hbm.at[wid], msg_v, sem).wait()
    plsc.subcore_barrier()
    pltpu.sync_copy(agg_sp.at[pl.ds(s * VS, VS)], buf_v)
    pltpu.sync_copy(buf_v, out_hbm.at[c].at[pl.ds(s * VS, VS)])


@functools.lru_cache(maxsize=1)
def _sc_calls():
    mesh = plsc.VectorSubcoreMesh(core_axis_name="c", subcore_axis_name="s",
                                  num_cores=NC, num_subcores=NS)
    params = pltpu.CompilerParams(use_tc_tiling_on_sc=False)
    gather = pl.kernel(
        _gather_body,
        out_type=jax.ShapeDtypeStruct((NW, K, B, D), F32),
        mesh=mesh,
        compiler_params=params,
        scratch_types=[
            pltpu.VMEM((K, B), jnp.int32),
            pltpu.VMEM((K, B, D), F32),
            pltpu.SemaphoreType.DMA,
        ],
    )
    scatter = pl.kernel(
        _scatter_body,
        out_type=jax.ShapeDtypeStruct((NC, V, D), F32),
        mesh=mesh,
        compiler_params=params,
        scratch_types=[
            pltpu.VMEM((K, B, D), F32),
            pltpu.VMEM((K, B), jnp.int32),
            pltpu.VMEM((VS, D), F32),
            pltpu.VMEM_SHARED((V, D), F32),
            pltpu.SemaphoreType.DMA,
        ],
    )
    return gather, scatter


def kernel(node_feats, edge_attr, edge_index, W_p, b_p, W_e1, b_e1, W_e2,
           b_e2, W_root, b_conv, W_ih, b_ih, W_hh, b_hh):
    d_in = node_feats.shape[1]
    d_e = edge_attr.shape[1]
    d_eh = W_e1.shape[1]
    eye8 = jnp.eye(8, dtype=F32)

    lanes = jnp.arange(D * D, dtype=jnp.int32)
    r_mat = (lanes[None, :] // D == jnp.arange(D, dtype=jnp.int32)[:, None]
             ).astype(F32)
    s_mat = (lanes[:, None] % D == jnp.arange(D, dtype=jnp.int32)[None, :]
             ).astype(F32)

    bf = jnp.bfloat16
    we1_8 = jnp.kron(eye8, W_e1).astype(bf)
    be1_8 = jnp.tile(b_e1, 8).reshape(1, 8 * d_eh).astype(bf)
    we2_8 = jnp.kron(eye8, W_e2).astype(bf)
    be2_8 = jnp.tile(b_e2, 8).reshape(1, 8 * D * D).astype(bf)
    r_8 = jnp.kron(eye8, r_mat).astype(bf)
    s_8 = jnp.kron(eye8, s_mat).astype(bf)
    wroot_8 = jnp.kron(eye8, W_root)
    bc_8 = jnp.tile(b_conv, 8).reshape(1, 128)
    wir_8 = jnp.kron(eye8, W_ih[0:D].T)
    wiz_8 = jnp.kron(eye8, W_ih[D:2 * D].T)
    win_8 = jnp.kron(eye8, W_ih[2 * D:3 * D].T)
    bi_8 = jnp.stack([jnp.tile(b_ih[0:D], 8), jnp.tile(b_ih[D:2 * D], 8),
                      jnp.tile(b_ih[2 * D:3 * D], 8)])
    whr_8 = jnp.kron(eye8, W_hh[0:D].T)
    whz_8 = jnp.kron(eye8, W_hh[D:2 * D].T)
    whn_8 = jnp.kron(eye8, W_hh[2 * D:3 * D].T)
    bh_8 = jnp.stack([jnp.tile(b_hh[0:D], 8), jnp.tile(b_hh[D:2 * D], 8),
                      jnp.tile(b_hh[2 * D:3 * D], 8)])

    src3 = edge_index[0].reshape(NH, NW, K, B)
    dst3 = edge_index[1].reshape(NH, NW, K, B)
    zeros_vs = jnp.zeros((VS, D), dtype=F32)

    RV = V // 8
    RE = E // 8
    RH = RE // NH
    ea8 = edge_attr.reshape(RE, 8 * d_e)

    proj = pl.pallas_call(
        _proj_body,
        out_shape=jax.ShapeDtypeStruct((RV, 128), F32),
    )
    x8 = proj(node_feats.reshape(RV, 8, d_in), W_p, b_p.reshape(1, D))

    T = 1000

    def make_msg_call(h):
        off = h * (RH // T)
        return pl.pallas_call(
            _msg_body,
            grid=(RH // T,),
            in_specs=[
                pl.BlockSpec((T, 8 * d_e), lambda i: (i + off, 0)),
                pl.BlockSpec((T, 128), lambda i: (i, 0)),
                pl.BlockSpec((8 * d_e, 8 * d_eh), lambda i: (0, 0)),
                pl.BlockSpec((1, 8 * d_eh), lambda i: (0, 0)),
                pl.BlockSpec((8 * d_eh, 8 * D * D), lambda i: (0, 0)),
                pl.BlockSpec((1, 8 * D * D), lambda i: (0, 0)),
                pl.BlockSpec((128, 8 * D * D), lambda i: (0, 0)),
                pl.BlockSpec((8 * D * D, 128), lambda i: (0, 0)),
            ],
            out_specs=(pl.BlockSpec((T, 128), lambda i: (i, 0)),
                       pl.BlockSpec((T, 8 * D * D), lambda i: (i, 0))),
            out_shape=(jax.ShapeDtypeStruct((RH, 128), F32),
                       jax.ShapeDtypeStruct((RH, 8 * D * D), bf)),
        )

    msg_cached_call = pl.pallas_call(
        _msg_cached_body,
        grid=(RH // T,),
        in_specs=[
            pl.BlockSpec((T, 128), lambda i: (i, 0)),
            pl.BlockSpec((T, 8 * D * D), lambda i: (i, 0)),
            pl.BlockSpec((128, 8 * D * D), lambda i: (0, 0)),
            pl.BlockSpec((8 * D * D, 128), lambda i: (0, 0)),
        ],
        out_specs=pl.BlockSpec((T, 128), lambda i: (i, 0)),
        out_shape=jax.ShapeDtypeStruct((RH, 128), F32),
    )
    msg_calls = [make_msg_call(h) for h in range(NH)]

    update_call = pl.pallas_call(
        _update_body,
        out_shape=jax.ShapeDtypeStruct((RV, 128), F32),
    )

    gather_call, scatter_call = _sc_calls()
    w8c = [None] * NH
    for step in range(3):
        xv = x8.reshape(V, D)
        xs = [gather_call(xv, src3[h]) for h in range(NH)]
        msg8 = [None] * NH
        for h in range(NH):
            if step == 0:
                msg8[h], w8c[h] = msg_calls[h](
                    ea8, xs[h].reshape(RH, 128), we1_8, be1_8,
                    we2_8, be2_8, r_8, s_8)
            else:
                msg8[h] = msg_cached_call(xs[h].reshape(RH, 128), w8c[h],
                                          r_8, s_8)
        parts = [scatter_call(msg8[h].reshape(NW, K, B, D), dst3[h], zeros_vs)
                 for h in range(NH)]
        x8 = update_call(parts[0].reshape(NC, RV, 128),
                         parts[1].reshape(NC, RV, 128), x8, wroot_8, bc_8,
                         wir_8, wiz_8, win_8, bi_8,
                         whr_8, whz_8, whn_8, bh_8)
    return (x8.reshape(V, D), edge_attr)


# speedup vs baseline: 1.0433x; 1.0433x over previous
import functools

import jax
import jax.numpy as jnp
from jax import lax
from jax.experimental import pallas as pl
from jax.experimental.pallas import tpu as pltpu
from jax.experimental.pallas import tpu_sc as plsc

F32 = jnp.float32

V = 10000
E = 160000
D = 16
NC = 2
NS = 16
NW = NC * NS
EW = E // NW
B = 125
K = EW // B
VS = V // NS


def _proj_body(nf_ref, wp_ref, bp_ref, out_ref):
    for k in range(8):
        out_ref[:, D * k:D * (k + 1)] = jnp.maximum(
            jnp.dot(nf_ref[:, k, :], wp_ref[...], preferred_element_type=F32)
            + bp_ref[...], 0.0)


def _msg_body(ea_ref, xs_ref, we1_ref, be1_ref, we2_ref, be2_ref,
              r_ref, s_ref, out_ref, wout_ref):
    bf = jnp.bfloat16

    def mm(a, m_ref):
        return jnp.dot(a.astype(bf), m_ref[...], preferred_element_type=F32)

    zero = jnp.asarray(0.0, bf)
    e1 = jnp.maximum(mm(ea_ref[...], we1_ref).astype(bf) + be1_ref[...], zero)
    w = jnp.maximum(mm(e1, we2_ref).astype(bf) + be2_ref[...], zero)
    wout_ref[...] = w
    xs_rep = mm(xs_ref[...], r_ref).astype(bf)
    out_ref[...] = mm(w * xs_rep, s_ref)


def _msg_cached_body(xs_ref, w_ref, r_ref, s_ref, out_ref):
    bf = jnp.bfloat16

    def mm(a, m_ref):
        return jnp.dot(a.astype(bf), m_ref[...], preferred_element_type=F32)

    xs_rep = mm(xs_ref[...], r_ref).astype(bf)
    out_ref[...] = mm(w_ref[...] * xs_rep, s_ref)


def _update_body(parts_ref, x_ref, wr_ref, bc_ref,
                 wir_ref, wiz_ref, win_ref, bi_ref,
                 whr_ref, whz_ref, whn_ref, bh_ref, out_ref):

    def mm(a, m_ref):
        return jnp.dot(a, m_ref[...], preferred_element_type=F32)

    x = x_ref[...]
    agg = parts_ref[0] + parts_ref[1]
    c = jnp.maximum(agg + mm(x, wr_ref) + bc_ref[...], 0.0)
    r = jax.nn.sigmoid(mm(c, wir_ref) + bi_ref[0:1, :]
                       + mm(x, whr_ref) + bh_ref[0:1, :])
    z = jax.nn.sigmoid(mm(c, wiz_ref) + bi_ref[1:2, :]
                       + mm(x, whz_ref) + bh_ref[1:2, :])
    n = jnp.tanh(mm(c, win_ref) + bi_ref[2:3, :]
                 + r * (mm(x, whn_ref) + bh_ref[2:3, :]))
    out_ref[...] = (1.0 - z) * n + z * x


def _gather_body(x_hbm, src_hbm, out_hbm, idx_v, rows_v, sem):
    c = lax.axis_index("c")
    s = lax.axis_index("s")
    wid = s * NC + c
    pltpu.sync_copy(src_hbm.at[wid], idx_v)

    def fire(j, carry):
        pltpu.async_copy(x_hbm.at[idx_v.at[j]], rows_v.at[j], sem)
        return carry

    lax.fori_loop(0, K, fire, 0)
    pltpu.make_async_copy(out_hbm.at[wid], rows_v, sem).wait()
    pltpu.sync_copy(rows_v, out_hbm.at[wid])


def _scatter_body(msg_hbm, dst_hbm, zeros_hbm, out_hbm,
                  msg_v, dst_v, buf_v, agg_sp, sem):
    c = lax.axis_index("c")
    s = lax.axis_index("s")
    wid = s * NC + c
    pltpu.sync_copy(zeros_hbm, buf_v)
    pltpu.sync_copy(buf_v, agg_sp.at[pl.ds(s * VS, VS)])
    pltpu.sync_copy(msg_hbm.at[wid], msg_v)
    pltpu.sync_copy(dst_hbm.at[wid], dst_v)
    plsc.subcore_barrier()

    def fire(j, carry):
        pltpu.async_copy(msg_v.at[j], agg_sp.at[dst_v.at[j]], sem, add=True)
        return carry

    lax.fori_loop(0, K, fire, 0)
    pltpu.make_async_copy(msg_hbm.at[wid], msg_v, sem).wait()
    plsc.subcore_barrier()
    pltpu.sync_copy(agg_sp.at[pl.ds(s * VS, VS)], buf_v)
    pltpu.sync_copy(buf_v, out_hbm.at[c].at[pl.ds(s * VS, VS)])


@functools.lru_cache(maxsize=1)
def _sc_calls():
    mesh = plsc.VectorSubcoreMesh(core_axis_name="c", subcore_axis_name="s",
                                  num_cores=NC, num_subcores=NS)
    params = pltpu.CompilerParams(use_tc_tiling_on_sc=False)
    gather = pl.kernel(
        _gather_body,
        out_type=jax.ShapeDtypeStruct((NW, K, B, D), F32),
        mesh=mesh,
        compiler_params=params,
        scratch_types=[
            pltpu.VMEM((K, B), jnp.int32),
            pltpu.VMEM((K, B, D), F32),
            pltpu.SemaphoreType.DMA,
        ],
    )
    scatter = pl.kernel(
        _scatter_body,
        out_type=jax.ShapeDtypeStruct((NC, V, D), F32),
        mesh=mesh,
        compiler_params=params,
        scratch_types=[
            pltpu.VMEM((K, B, D), F32),
            pltpu.VMEM((K, B), jnp.int32),
            pltpu.VMEM((VS, D), F32),
            pltpu.VMEM_SHARED((V, D), F32),
            pltpu.SemaphoreType.DMA,
        ],
    )
    return gather, scatter


def kernel(node_feats, edge_attr, edge_index, W_p, b_p, W_e1, b_e1, W_e2,
           b_e2, W_root, b_conv, W_ih, b_ih, W_hh, b_hh):
    d_in = node_feats.shape[1]
    d_e = edge_attr.shape[1]
    d_eh = W_e1.shape[1]
    eye8 = jnp.eye(8, dtype=F32)

    lanes = jnp.arange(D * D, dtype=jnp.int32)
    r_mat = (lanes[None, :] // D == jnp.arange(D, dtype=jnp.int32)[:, None]
             ).astype(F32)
    s_mat = (lanes[:, None] % D == jnp.arange(D, dtype=jnp.int32)[None, :]
             ).astype(F32)

    bf = jnp.bfloat16
    we1_8 = jnp.kron(eye8, W_e1).astype(bf)
    be1_8 = jnp.tile(b_e1, 8).reshape(1, 8 * d_eh).astype(bf)
    we2_8 = jnp.kron(eye8, W_e2).astype(bf)
    be2_8 = jnp.tile(b_e2, 8).reshape(1, 8 * D * D).astype(bf)
    r_8 = jnp.kron(eye8, r_mat).astype(bf)
    s_8 = jnp.kron(eye8, s_mat).astype(bf)
    wroot_8 = jnp.kron(eye8, W_root)
    bc_8 = jnp.tile(b_conv, 8).reshape(1, 128)
    wir_8 = jnp.kron(eye8, W_ih[0:D].T)
    wiz_8 = jnp.kron(eye8, W_ih[D:2 * D].T)
    win_8 = jnp.kron(eye8, W_ih[2 * D:3 * D].T)
    bi_8 = jnp.stack([jnp.tile(b_ih[0:D], 8), jnp.tile(b_ih[D:2 * D], 8),
                      jnp.tile(b_ih[2 * D:3 * D], 8)])
    whr_8 = jnp.kron(eye8, W_hh[0:D].T)
    whz_8 = jnp.kron(eye8, W_hh[D:2 * D].T)
    whn_8 = jnp.kron(eye8, W_hh[2 * D:3 * D].T)
    bh_8 = jnp.stack([jnp.tile(b_hh[0:D], 8), jnp.tile(b_hh[D:2 * D], 8),
                      jnp.tile(b_hh[2 * D:3 * D], 8)])

    src3 = edge_index[0].reshape(NW, K, B)
    dst3 = edge_index[1].reshape(NW, K, B)
    zeros_vs = jnp.zeros((VS, D), dtype=F32)

    RV = V // 8
    RE = E // 8
    ea8 = edge_attr.reshape(RE, 8 * d_e)

    proj = pl.pallas_call(
        _proj_body,
        out_shape=jax.ShapeDtypeStruct((RV, 128), F32),
    )
    x8 = proj(node_feats.reshape(RV, 8, d_in), W_p, b_p.reshape(1, D))

    T = 1000
    msg_call = pl.pallas_call(
        _msg_body,
        grid=(RE // T,),
        in_specs=[
            pl.BlockSpec((T, 8 * d_e), lambda i: (i, 0)),
            pl.BlockSpec((T, 128), lambda i: (i, 0)),
            pl.BlockSpec((8 * d_e, 8 * d_eh), lambda i: (0, 0)),
            pl.BlockSpec((1, 8 * d_eh), lambda i: (0, 0)),
            pl.BlockSpec((8 * d_eh, 8 * D * D), lambda i: (0, 0)),
            pl.BlockSpec((1, 8 * D * D), lambda i: (0, 0)),
            pl.BlockSpec((128, 8 * D * D), lambda i: (0, 0)),
            pl.BlockSpec((8 * D * D, 128), lambda i: (0, 0)),
        ],
        out_specs=(pl.BlockSpec((T, 128), lambda i: (i, 0)),
                   pl.BlockSpec((T, 8 * D * D), lambda i: (i, 0))),
        out_shape=(jax.ShapeDtypeStruct((RE, 128), F32),
                   jax.ShapeDtypeStruct((RE, 8 * D * D), bf)),
    )

    TC2 = 2000
    msg_cached_call = pl.pallas_call(
        _msg_cached_body,
        grid=(RE // TC2,),
        in_specs=[
            pl.BlockSpec((TC2, 128), lambda i: (i, 0)),
            pl.BlockSpec((TC2, 8 * D * D), lambda i: (i, 0)),
            pl.BlockSpec((128, 8 * D * D), lambda i: (0, 0)),
            pl.BlockSpec((8 * D * D, 128), lambda i: (0, 0)),
        ],
        out_specs=pl.BlockSpec((TC2, 128), lambda i: (i, 0)),
        out_shape=jax.ShapeDtypeStruct((RE, 128), F32),
    )

    update_call = pl.pallas_call(
        _update_body,
        out_shape=jax.ShapeDtypeStruct((RV, 128), F32),
    )

    gather_call, scatter_call = _sc_calls()
    w8c = None
    for step in range(3):
        xs = gather_call(x8.reshape(V, D), src3)
        if step == 0:
            msg8, w8c = msg_call(ea8, xs.reshape(RE, 128), we1_8, be1_8,
                                 we2_8, be2_8, r_8, s_8)
        else:
            msg8 = msg_cached_call(xs.reshape(RE, 128), w8c, r_8, s_8)
        parts = scatter_call(msg8.reshape(NW, K, B, D), dst3, zeros_vs)
        x8 = update_call(parts.reshape(NC, RV, 128), x8, wroot_8, bc_8,
                         wir_8, wiz_8, win_8, bi_8,
                         whr_8, whz_8, whn_8, bh_8)
    return (x8.reshape(V, D), edge_attr)


# speedup vs baseline: 1.0463x; 1.0028x over previous
import functools

import jax
import jax.numpy as jnp
from jax import lax
from jax.experimental import pallas as pl
from jax.experimental.pallas import tpu as pltpu
from jax.experimental.pallas import tpu_sc as plsc

F32 = jnp.float32

V = 10000
E = 160000
D = 16
NC = 2
NS = 16
NW = NC * NS
EW = E // NW
B = 125
K = EW // B
VS = V // NS


def _proj_body(nf_ref, wp_ref, bp_ref, out_ref):
    for k in range(8):
        out_ref[:, D * k:D * (k + 1)] = jnp.maximum(
            jnp.dot(nf_ref[:, k, :], wp_ref[...], preferred_element_type=F32)
            + bp_ref[...], 0.0)


def _msg_body(ea_ref, xs_ref, we1_ref, be1_ref, we2_ref, be2_ref,
              r_ref, s_ref, out_ref, wout_ref):
    bf = jnp.bfloat16

    def mm(a, m_ref):
        return jnp.dot(a.astype(bf), m_ref[...], preferred_element_type=F32)

    zero = jnp.asarray(0.0, bf)
    e1 = jnp.maximum(mm(ea_ref[...], we1_ref).astype(bf) + be1_ref[...], zero)
    w = jnp.maximum(mm(e1, we2_ref).astype(bf) + be2_ref[...], zero)
    wout_ref[...] = w
    xs_rep = mm(xs_ref[...], r_ref).astype(bf)
    out_ref[...] = mm(w * xs_rep, s_ref)


def _msg_cached_body(xs_ref, w_ref, r_ref, s_ref, out_ref):
    bf = jnp.bfloat16

    def mm(a, m_ref):
        return jnp.dot(a.astype(bf), m_ref[...], preferred_element_type=F32)

    xs_rep = mm(xs_ref[...], r_ref).astype(bf)
    out_ref[...] = mm(w_ref[...] * xs_rep, s_ref)


def _update_body(parts_ref, x_ref, wr_ref, bc_ref,
                 wir_ref, wiz_ref, win_ref, bi_ref,
                 whr_ref, whz_ref, whn_ref, bh_ref, out_ref):

    def mm(a, m_ref):
        return jnp.dot(a, m_ref[...], preferred_element_type=F32)

    x = x_ref[...]
    agg = parts_ref[0] + parts_ref[1]
    c = jnp.maximum(agg + mm(x, wr_ref) + bc_ref[...], 0.0)
    r = jax.nn.sigmoid(mm(c, wir_ref) + bi_ref[0:1, :]
                       + mm(x, whr_ref) + bh_ref[0:1, :])
    z = jax.nn.sigmoid(mm(c, wiz_ref) + bi_ref[1:2, :]
                       + mm(x, whz_ref) + bh_ref[1:2, :])
    n = jnp.tanh(mm(c, win_ref) + bi_ref[2:3, :]
                 + r * (mm(x, whn_ref) + bh_ref[2:3, :]))
    out_ref[...] = (1.0 - z) * n + z * x


def _gather_body(x_hbm, src_hbm, out_hbm, idx_v, rows_v, sem):
    c = lax.axis_index("c")
    s = lax.axis_index("s")
    wid = s * NC + c
    pltpu.sync_copy(src_hbm.at[wid], idx_v)

    def fire(j, carry):
        pltpu.async_copy(x_hbm.at[idx_v.at[j]], rows_v.at[j], sem)
        return carry

    lax.fori_loop(0, K, fire, 0)
    pltpu.make_async_copy(out_hbm.at[wid], rows_v, sem).wait()
    pltpu.sync_copy(rows_v, out_hbm.at[wid])


def _scatter_body(msg_hbm, dst_hbm, zeros_hbm, out_hbm,
                  msg_v, dst_v, buf_v, agg_sp, sem):
    c = lax.axis_index("c")
    s = lax.axis_index("s")
    wid = s * NC + c
    pltpu.sync_copy(zeros_hbm, buf_v)
    pltpu.sync_copy(buf_v, agg_sp.at[pl.ds(s * VS, VS)])
    pltpu.sync_copy(msg_hbm.at[wid], msg_v)
    pltpu.sync_copy(dst_hbm.at[wid], dst_v)
    plsc.subcore_barrier()

    def fire(j, carry):
        pltpu.async_copy(msg_v.at[j], agg_sp.at[dst_v.at[j]], sem, add=True)
        return carry

    lax.fori_loop(0, K, fire, 0)
    pltpu.make_async_copy(msg_hbm.at[wid], msg_v, sem).wait()
    plsc.subcore_barrier()
    pltpu.sync_copy(agg_sp.at[pl.ds(s * VS, VS)], buf_v)
    pltpu.sync_copy(buf_v, out_hbm.at[c].at[pl.ds(s * VS, VS)])


@functools.lru_cache(maxsize=1)
def _sc_calls():
    mesh = plsc.VectorSubcoreMesh(core_axis_name="c", subcore_axis_name="s",
                                  num_cores=NC, num_subcores=NS)
    params = pltpu.CompilerParams(use_tc_tiling_on_sc=False)
    gather = pl.kernel(
        _gather_body,
        out_type=jax.ShapeDtypeStruct((NW, K, B, D), F32),
        mesh=mesh,
        compiler_params=params,
        scratch_types=[
            pltpu.VMEM((K, B), jnp.int32),
            pltpu.VMEM((K, B, D), F32),
            pltpu.SemaphoreType.DMA,
        ],
    )
    scatter = pl.kernel(
        _scatter_body,
        out_type=jax.ShapeDtypeStruct((NC, V, D), F32),
        mesh=mesh,
        compiler_params=params,
        scratch_types=[
            pltpu.VMEM((K, B, D), F32),
            pltpu.VMEM((K, B), jnp.int32),
            pltpu.VMEM((VS, D), F32),
            pltpu.VMEM_SHARED((V, D), F32),
            pltpu.SemaphoreType.DMA,
        ],
    )
    return gather, scatter


def kernel(node_feats, edge_attr, edge_index, W_p, b_p, W_e1, b_e1, W_e2,
           b_e2, W_root, b_conv, W_ih, b_ih, W_hh, b_hh):
    d_in = node_feats.shape[1]
    d_e = edge_attr.shape[1]
    d_eh = W_e1.shape[1]
    eye8 = jnp.eye(8, dtype=F32)

    lanes = jnp.arange(D * D, dtype=jnp.int32)
    r_mat = (lanes[None, :] // D == jnp.arange(D, dtype=jnp.int32)[:, None]
             ).astype(F32)
    s_mat = (lanes[:, None] % D == jnp.arange(D, dtype=jnp.int32)[None, :]
             ).astype(F32)

    bf = jnp.bfloat16
    we1_8 = jnp.kron(eye8, W_e1).astype(bf)
    be1_8 = jnp.tile(b_e1, 8).reshape(1, 8 * d_eh).astype(bf)
    we2_8 = jnp.kron(eye8, W_e2).astype(bf)
    be2_8 = jnp.tile(b_e2, 8).reshape(1, 8 * D * D).astype(bf)
    r_8 = jnp.kron(eye8, r_mat).astype(bf)
    s_8 = jnp.kron(eye8, s_mat).astype(bf)
    wroot_8 = jnp.kron(eye8, W_root)
    bc_8 = jnp.tile(b_conv, 8).reshape(1, 128)
    wir_8 = jnp.kron(eye8, W_ih[0:D].T)
    wiz_8 = jnp.kron(eye8, W_ih[D:2 * D].T)
    win_8 = jnp.kron(eye8, W_ih[2 * D:3 * D].T)
    bi_8 = jnp.stack([jnp.tile(b_ih[0:D], 8), jnp.tile(b_ih[D:2 * D], 8),
                      jnp.tile(b_ih[2 * D:3 * D], 8)])
    whr_8 = jnp.kron(eye8, W_hh[0:D].T)
    whz_8 = jnp.kron(eye8, W_hh[D:2 * D].T)
    whn_8 = jnp.kron(eye8, W_hh[2 * D:3 * D].T)
    bh_8 = jnp.stack([jnp.tile(b_hh[0:D], 8), jnp.tile(b_hh[D:2 * D], 8),
                      jnp.tile(b_hh[2 * D:3 * D], 8)])

    src3 = edge_index[0].reshape(NW, K, B)
    dst3 = edge_index[1].reshape(NW, K, B)
    zeros_vs = jnp.zeros((VS, D), dtype=F32)

    RV = V // 8
    RE = E // 8
    ea8 = edge_attr.reshape(RE, 8 * d_e)

    proj = pl.pallas_call(
        _proj_body,
        out_shape=jax.ShapeDtypeStruct((RV, 128), F32),
    )
    x8 = proj(node_feats.reshape(RV, 8, d_in), W_p, b_p.reshape(1, D))

    T = 2000
    msg_call = pl.pallas_call(
        _msg_body,
        grid=(RE // T,),
        in_specs=[
            pl.BlockSpec((T, 8 * d_e), lambda i: (i, 0)),
            pl.BlockSpec((T, 128), lambda i: (i, 0)),
            pl.BlockSpec((8 * d_e, 8 * d_eh), lambda i: (0, 0)),
            pl.BlockSpec((1, 8 * d_eh), lambda i: (0, 0)),
            pl.BlockSpec((8 * d_eh, 8 * D * D), lambda i: (0, 0)),
            pl.BlockSpec((1, 8 * D * D), lambda i: (0, 0)),
            pl.BlockSpec((128, 8 * D * D), lambda i: (0, 0)),
            pl.BlockSpec((8 * D * D, 128), lambda i: (0, 0)),
        ],
        out_specs=(pl.BlockSpec((T, 128), lambda i: (i, 0)),
                   pl.BlockSpec((T, 8 * D * D), lambda i: (i, 0))),
        out_shape=(jax.ShapeDtypeStruct((RE, 128), F32),
                   jax.ShapeDtypeStruct((RE, 8 * D * D), bf)),
    )

    TC2 = 2000
    msg_cached_call = pl.pallas_call(
        _msg_cached_body,
        grid=(RE // TC2,),
        in_specs=[
            pl.BlockSpec((TC2, 128), lambda i: (i, 0)),
            pl.BlockSpec((TC2, 8 * D * D), lambda i: (i, 0)),
            pl.BlockSpec((128, 8 * D * D), lambda i: (0, 0)),
            pl.BlockSpec((8 * D * D, 128), lambda i: (0, 0)),
        ],
        out_specs=pl.BlockSpec((TC2, 128), lambda i: (i, 0)),
        out_shape=jax.ShapeDtypeStruct((RE, 128), F32),
    )

    update_call = pl.pallas_call(
        _update_body,
        out_shape=jax.ShapeDtypeStruct((RV, 128), F32),
    )

    gather_call, scatter_call = _sc_calls()
    w8c = None
    for step in range(3):
        xs = gather_call(x8.reshape(V, D), src3)
        if step == 0:
            msg8, w8c = msg_call(ea8, xs.reshape(RE, 128), we1_8, be1_8,
                                 we2_8, be2_8, r_8, s_8)
        else:
            msg8 = msg_cached_call(xs.reshape(RE, 128), w8c, r_8, s_8)
        parts = scatter_call(msg8.reshape(NW, K, B, D), dst3, zeros_vs)
        x8 = update_call(parts.reshape(NC, RV, 128), x8, wroot_8, bc_8,
                         wir_8, wiz_8, win_8, bi_8,
                         whr_8, whz_8, whn_8, bh_8)
    return (x8.reshape(V, D), edge_attr)
